# baseline (speedup 1.0000x reference)
import functools

import jax
import jax.numpy as jnp
from jax.experimental import pallas as pl
from jax.experimental.pallas import tpu as pltpu

BLOCK_TOKENS = 1024


def _router_kernel(x_ref, w_ref, logits_ref, probs_ref):
    x = x_ref[...].astype(jnp.bfloat16)
    w = w_ref[...].astype(jnp.bfloat16)
    logits = jax.lax.dot_general(
        w, x, (((1,), (1,)), ((), ())), preferred_element_type=jnp.float32
    )
    m = jnp.max(logits, axis=0, keepdims=True)
    e = jnp.exp(logits - m)
    probs = e / jnp.sum(e, axis=0, keepdims=True)
    logits_ref[...] = logits
    probs_ref[...] = probs


@functools.partial(jax.jit, static_argnames=())
def kernel(hidden_states, W):
    tokens, hidden = hidden_states.shape
    num_experts = W.shape[0]
    grid = (tokens // BLOCK_TOKENS,)
    out_shape = jax.ShapeDtypeStruct((num_experts, tokens), jnp.float32)
    logits_t, probs_t = pl.pallas_call(
        _router_kernel,
        grid=grid,
        in_specs=[
            pl.BlockSpec((BLOCK_TOKENS, hidden), lambda i: (i, 0)),
            pl.BlockSpec((num_experts, hidden), lambda i: (0, 0)),
        ],
        out_specs=[
            pl.BlockSpec((num_experts, BLOCK_TOKENS), lambda i: (0, i)),
            pl.BlockSpec((num_experts, BLOCK_TOKENS), lambda i: (0, i)),
        ],
        out_shape=[out_shape, out_shape],
        compiler_params=pltpu.CompilerParams(
            dimension_semantics=("arbitrary",),
        ),
    )(hidden_states, W)
    return (logits_t.T, probs_t.T)
